# Initial kernel scaffold; baseline (speedup 1.0000x reference)
import jax
import jax.numpy as jnp
from jax.experimental import pallas as pl


def kernel(points, logits):
    raise NotImplementedError("write your pallas kernel here")


# speedup vs baseline: 14.8772x; 14.8772x over previous
import jax
import jax.numpy as jnp
from jax.experimental import pallas as pl
from jax.experimental.pallas import tpu as pltpu

P = 4096
K = 16
C = 13
RADIUS2 = 0.01
RB = 256


def _lane_cumsum(x):
    n = x.shape[-1]
    lane = jax.lax.broadcasted_iota(jnp.int32, x.shape, len(x.shape) - 1)
    s = 1
    while s < n:
        shifted = pltpu.roll(x, s, axis=len(x.shape) - 1)
        x = x + jnp.where(lane >= s, shifted, 0.0)
        s *= 2
    return x


def _tv_body(pts_r_ref, ptsT_ref, log_r_ref, logT_ref, out_ref):
    n = pl.program_id(0)
    i = pl.program_id(1)

    x = pts_r_ref[0]
    xT = ptsT_ref[0]
    d2 = jnp.zeros((RB, P), jnp.float32)
    for d in range(3):
        t = x[:, d:d + 1] - xT[d:d + 1, :]
        d2 = d2 + t * t
    within = (d2 < RADIUS2).astype(jnp.float32)

    rank = _lane_cumsum(within)
    sel = within * (rank <= K).astype(jnp.float32)

    lg = log_r_ref[0]
    lT = logT_ref[0]
    w = jnp.zeros((RB, P), jnp.float32)
    for c in range(C):
        w = w + jnp.abs(lT[c:c + 1, :] - lg[:, c:c + 1])

    S = jnp.sum(sel * w, axis=1)
    total = jnp.sum(within, axis=1)
    length = jnp.minimum(total, float(K))
    m = jnp.sum(jnp.abs(lg), axis=1)
    tv = (S + (K - length) * m) / (C * length)
    partial = jnp.sum(tv)

    first = jnp.logical_and(n == 0, i == 0)

    partial2d = partial.reshape(1, 1)

    @pl.when(first)
    def _():
        out_ref[...] = partial2d

    @pl.when(jnp.logical_not(first))
    def _():
        out_ref[...] = out_ref[...] + partial2d


def kernel(points, logits):
    N = points.shape[0]
    ptsT = points.transpose(0, 2, 1)
    logT = logits.transpose(0, 2, 1)
    out = pl.pallas_call(
        _tv_body,
        grid=(N, P // RB),
        in_specs=[
            pl.BlockSpec((1, RB, 3), lambda n, i: (n, i, 0)),
            pl.BlockSpec((1, 3, P), lambda n, i: (n, 0, 0)),
            pl.BlockSpec((1, RB, C), lambda n, i: (n, i, 0)),
            pl.BlockSpec((1, C, P), lambda n, i: (n, 0, 0)),
        ],
        out_specs=pl.BlockSpec((1, 1), lambda n, i: (0, 0)),
        out_shape=jax.ShapeDtypeStruct((1, 1), jnp.float32),
    )(points, ptsT, logits, logT)
    return out[0, 0] / (N * P)
